# baseline (speedup 1.0000x reference)
import functools

import jax
import jax.numpy as jnp
from jax import lax
from jax.experimental import pallas as pl
from jax.experimental.pallas import tpu as pltpu
from jax.experimental.pallas import tpu_sc as plsc

B, L, D = 16, 4096, 1024
NC, NS = 2, 16
NW = NC * NS
HALF = L // 2
CHR = 32
NCHUNK = HALF // CHR


def _sc_body(x_hbm, tok_hbm, params_hbm, out_hbm,
             buf, row16_v, prow_v, idx2_v, idx16_v, sem_in, sem_out, sem_m):
    wid = lax.axis_index("s") * NC + lax.axis_index("c")

    pltpu.sync_copy(
        params_hbm.at[pl.ds(pl.multiple_of(wid * 8, 8), 8)], prow_v)
    p = prow_v[0]
    in0, out0, tok_row, own_tok, wrap_src, wrap_dst, do_wrap = (
        p[0], p[1], p[2], p[3], p[4], p[5], p[6])

    lane = lax.broadcasted_iota(jnp.int32, (NS,), 0)

    loads = [None, None]
    stores = [None, None]
    loads[0] = pltpu.async_copy(
        x_hbm.at[pl.ds(pl.multiple_of(in0, 8), CHR)], buf.at[0],
        sem_in.at[0])
    for i in range(NCHUNK):
        cur = i & 1
        nxt = cur ^ 1
        if i < NCHUNK - 1:
            if i >= 1:
                stores[nxt].wait()
            loads[nxt] = pltpu.async_copy(
                x_hbm.at[pl.ds(pl.multiple_of(in0 + (i + 1) * CHR, 8), CHR)],
                buf.at[nxt], sem_in.at[nxt])
        loads[cur].wait()
        base = out0 + i * CHR
        idx2_v[cur, 0:NS] = base + lane
        idx2_v[cur, NS:2 * NS] = base + NS + lane
        stores[cur] = pltpu.async_copy(
            buf.at[cur], out_hbm.at[idx2_v.at[cur]], sem_out.at[cur])
    stores[0].wait()
    stores[1].wait()

    @pl.when(own_tok == 1)
    def _token():
        for j in range(NS):
            pltpu.sync_copy(tok_hbm, row16_v.at[pl.ds(j, 1)])
        idx16_v[...] = jnp.full((NS,), tok_row, jnp.int32)
        pltpu.async_copy(row16_v, out_hbm.at[idx16_v], sem_m).wait()

    @pl.when(do_wrap == 1)
    def _wrap():
        for j in range(NS):
            pltpu.sync_copy(x_hbm.at[pl.ds(pl.multiple_of(wrap_src, 8), 1)],
                            row16_v.at[pl.ds(j, 1)])
        idx16_v[...] = jnp.full((NS,), wrap_dst, jnp.int32)
        pltpu.async_copy(row16_v, out_hbm.at[idx16_v], sem_m).wait()


def _ts_body(lens_ref, ts_ref, out_ts_ref, out_len_ref):
    cols = lax.broadcasted_iota(jnp.int32, (1, L), 1)
    for b in range(B):
        last = lens_ref[b]
        last_m1 = jnp.maximum(last - 1, 0)
        row = ts_ref[b:b + 1, :]
        last_ts = jnp.sum(jnp.where(cols == last_m1, row, 0.0))
        out_ts_ref[b:b + 1, :L] = jnp.where(cols == last, last_ts, row)
        out_ts_ref[b:b + 1, L:L + 1] = row[:, 0:1]
        out_len_ref[b] = last + 1


def kernel(x_payload, timestamps, seq_lens, token):
    seq_lens = seq_lens.astype(jnp.int32)
    token2 = token.reshape(1, D)

    w = jnp.arange(NW, dtype=jnp.int32)
    wb = w // 2
    wh = w % 2
    wlast = seq_lens[wb]
    params = jnp.zeros((NW * 8, 16), jnp.int32)
    params = params.at[w * 8, 0].set(wb * L + wh * HALF)
    params = params.at[w * 8, 1].set(wb * (L + 1) + wh * HALF)
    params = params.at[w * 8, 2].set(wb * (L + 1) + wlast)
    params = params.at[w * 8, 3].set(
        ((wlast >= wh * HALF) & (wlast < wh * HALF + HALF)).astype(jnp.int32))
    params = params.at[w * 8, 4].set(wb * L)
    params = params.at[w * 8, 5].set(wb * (L + 1) + L)
    params = params.at[w * 8, 6].set(wh)

    x2d = x_payload.reshape(B * L, D)

    mesh = plsc.VectorSubcoreMesh(core_axis_name="c", subcore_axis_name="s")
    sc_copy = functools.partial(
        pl.kernel,
        out_type=jax.ShapeDtypeStruct((B * (L + 1), D), x_payload.dtype),
        mesh=mesh,
        scratch_types=[
            pltpu.VMEM((2, CHR, D), jnp.float32),
            pltpu.VMEM((NS, D), jnp.float32),
            pltpu.VMEM((8, 16), jnp.int32),
            pltpu.VMEM((2, 2 * NS), jnp.int32),
            pltpu.VMEM((NS,), jnp.int32),
            pltpu.SemaphoreType.DMA((2,)),
            pltpu.SemaphoreType.DMA((2,)),
            pltpu.SemaphoreType.DMA,
        ],
    )(_sc_body)
    new_x = sc_copy(x2d, token2, params).reshape(B, L + 1, D)

    new_ts, new_len = pl.pallas_call(
        _ts_body,
        in_specs=[
            pl.BlockSpec(memory_space=pltpu.SMEM),
            pl.BlockSpec(memory_space=pltpu.VMEM),
        ],
        out_specs=[
            pl.BlockSpec(memory_space=pltpu.VMEM),
            pl.BlockSpec(memory_space=pltpu.SMEM),
        ],
        out_shape=[
            jax.ShapeDtypeStruct((B, L + 1), timestamps.dtype),
            jax.ShapeDtypeStruct((B,), jnp.int32),
        ],
    )(seq_lens, timestamps)
    return new_x, new_len, new_ts, new_len


# speedup vs baseline: 1.4746x; 1.4746x over previous
import functools

import jax
import jax.numpy as jnp
from jax import lax
from jax.experimental import pallas as pl
from jax.experimental.pallas import tpu as pltpu
from jax.experimental.pallas import tpu_sc as plsc

B, L, D = 16, 4096, 1024
NC, NS = 2, 16
NW = NC * NS
HALF = L // 2
CHR = 32
NCHUNK = HALF // CHR


def _sc_body(x_hbm, params_hbm, out_hbm,
             buf, row_v, prow_v, sem_in, sem_out, sem_m):
    wid = lax.axis_index("s") * NC + lax.axis_index("c")

    pltpu.sync_copy(
        params_hbm.at[pl.ds(pl.multiple_of(wid * 8, 8), 8)], prow_v)
    p = prow_v[0]
    b, row0, do_wrap = p[0], p[1], p[2]

    loads = [None, None]
    stores = [None, None]
    loads[0] = pltpu.async_copy(
        x_hbm.at[b, pl.ds(pl.multiple_of(row0, 8), CHR)], buf.at[0],
        sem_in.at[0])
    for i in range(NCHUNK):
        cur = i & 1
        nxt = cur ^ 1
        if i < NCHUNK - 1:
            if i >= 1:
                stores[nxt].wait()
            loads[nxt] = pltpu.async_copy(
                x_hbm.at[b, pl.ds(pl.multiple_of(row0 + (i + 1) * CHR, 8),
                                  CHR)],
                buf.at[nxt], sem_in.at[nxt])
        loads[cur].wait()
        stores[cur] = pltpu.async_copy(
            buf.at[cur],
            out_hbm.at[b, pl.ds(pl.multiple_of(row0 + i * CHR, 8), CHR)],
            sem_out.at[cur])
    stores[0].wait()
    stores[1].wait()

    @pl.when(do_wrap == 1)
    def _wrap():
        pltpu.async_copy(x_hbm.at[b, pl.ds(0, 1)], row_v, sem_m).wait()
        pltpu.async_copy(row_v, out_hbm.at[b, pl.ds(L, 1)], sem_m).wait()


def _tok_body(lens_ref, x_ref, tok_ref, out_ref):
    b = pl.program_id(0)
    r = lens_ref[b] % 8
    rows = lax.broadcasted_iota(jnp.int32, (8, 1), 0)
    out_ref[0] = jnp.where(rows == r, tok_ref[...], x_ref[0])


def _ts_body(lens_ref, ts_ref, out_ts_ref, out_len_ref):
    cols = lax.broadcasted_iota(jnp.int32, (1, L), 1)
    for b in range(B):
        last = lens_ref[b]
        last_m1 = jnp.maximum(last - 1, 0)
        row = ts_ref[b:b + 1, :]
        last_ts = jnp.sum(jnp.where(cols == last_m1, row, 0.0))
        out_ts_ref[b:b + 1, :L] = jnp.where(cols == last, last_ts, row)
        out_ts_ref[b:b + 1, L:L + 1] = row[:, 0:1]
        out_len_ref[b] = last + 1


def kernel(x_payload, timestamps, seq_lens, token):
    seq_lens = seq_lens.astype(jnp.int32)
    token2 = token.reshape(1, D)

    w = jnp.arange(NW, dtype=jnp.int32)
    params = jnp.zeros((NW * 8, 16), jnp.int32)
    params = params.at[w * 8, 0].set(w // 2)
    params = params.at[w * 8, 1].set((w % 2) * HALF)
    params = params.at[w * 8, 2].set(w % 2)

    mesh = plsc.VectorSubcoreMesh(core_axis_name="c", subcore_axis_name="s")
    sc_copy = functools.partial(
        pl.kernel,
        out_type=jax.ShapeDtypeStruct((B, L + 1, D), x_payload.dtype),
        mesh=mesh,
        scratch_types=[
            pltpu.VMEM((2, CHR, D), jnp.float32),
            pltpu.VMEM((1, D), jnp.float32),
            pltpu.VMEM((8, 16), jnp.int32),
            pltpu.SemaphoreType.DMA((2,)),
            pltpu.SemaphoreType.DMA((2,)),
            pltpu.SemaphoreType.DMA,
        ],
    )(_sc_body)
    copied = sc_copy(x_payload, params)

    tok_grid = pltpu.PrefetchScalarGridSpec(
        num_scalar_prefetch=1,
        grid=(B,),
        in_specs=[
            pl.BlockSpec((1, 8, D), lambda b, lens: (b, lens[b] // 8, 0)),
            pl.BlockSpec((1, D), lambda b, lens: (0, 0)),
        ],
        out_specs=pl.BlockSpec((1, 8, D), lambda b, lens: (b, lens[b] // 8, 0)),
    )
    new_x = pl.pallas_call(
        _tok_body,
        grid_spec=tok_grid,
        out_shape=jax.ShapeDtypeStruct((B, L + 1, D), x_payload.dtype),
        input_output_aliases={1: 0},
    )(seq_lens, copied, token2)

    new_ts, new_len = pl.pallas_call(
        _ts_body,
        in_specs=[
            pl.BlockSpec(memory_space=pltpu.SMEM),
            pl.BlockSpec(memory_space=pltpu.VMEM),
        ],
        out_specs=[
            pl.BlockSpec(memory_space=pltpu.VMEM),
            pl.BlockSpec(memory_space=pltpu.SMEM),
        ],
        out_shape=[
            jax.ShapeDtypeStruct((B, L + 1), timestamps.dtype),
            jax.ShapeDtypeStruct((B,), jnp.int32),
        ],
    )(seq_lens, timestamps)
    return new_x, new_len, new_ts, new_len


# speedup vs baseline: 1.5025x; 1.0189x over previous
import functools

import jax
import jax.numpy as jnp
from jax import lax
from jax.experimental import pallas as pl
from jax.experimental.pallas import tpu as pltpu
from jax.experimental.pallas import tpu_sc as plsc

B, L, D = 16, 4096, 1024
NC, NS = 2, 16
NW = NC * NS
HALF = L // 2
CHR = 32
NCHUNK = HALF // CHR


def _sc_body(x_hbm, tok_hbm, params_hbm, out_hbm,
             buf, row_v, prow_v, sem_in, sem_out, sem_m):
    wid = lax.axis_index("s") * NC + lax.axis_index("c")
    b = wid // 2
    half = wid % 2
    row0 = half * HALF

    pltpu.sync_copy(
        params_hbm.at[pl.ds(pl.multiple_of(wid * 8, 8), 8)], prow_v)
    last = prow_v[0][0]
    own_tok = (last >= row0) & (last < row0 + HALF)
    offs = last - row0
    tok_chunk = offs // CHR
    tok_rloc = offs % CHR

    @pl.when(half == 1)
    def _wrap():
        pltpu.async_copy(x_hbm.at[b, pl.ds(0, 1)], row_v, sem_m).wait()
        pltpu.async_copy(row_v, out_hbm.at[b, pl.ds(L, 1)], sem_m).wait()

    loads = [None, None]
    stores = [None, None]
    loads[0] = pltpu.async_copy(
        x_hbm.at[b, pl.ds(pl.multiple_of(row0, 8), CHR)], buf.at[0],
        sem_in.at[0])
    for i in range(NCHUNK):
        cur = i & 1
        nxt = cur ^ 1
        if i < NCHUNK - 1:
            if i >= 1:
                stores[nxt].wait()
            loads[nxt] = pltpu.async_copy(
                x_hbm.at[b, pl.ds(pl.multiple_of(row0 + (i + 1) * CHR, 8),
                                  CHR)],
                buf.at[nxt], sem_in.at[nxt])
        loads[cur].wait()

        @pl.when(own_tok & (tok_chunk == i))
        def _patch(cur=cur):
            pltpu.sync_copy(tok_hbm, buf.at[cur, pl.ds(tok_rloc, 1)])

        stores[cur] = pltpu.async_copy(
            buf.at[cur],
            out_hbm.at[b, pl.ds(pl.multiple_of(row0 + i * CHR, 8), CHR)],
            sem_out.at[cur])
    stores[0].wait()
    stores[1].wait()


def _ts_body(lens_ref, ts_ref, out_ts_ref, out_len_ref):
    cols = lax.broadcasted_iota(jnp.int32, (1, L), 1)
    for b in range(B):
        last = lens_ref[b]
        last_m1 = jnp.maximum(last - 1, 0)
        row = ts_ref[b:b + 1, :]
        last_ts = jnp.sum(jnp.where(cols == last_m1, row, 0.0))
        out_ts_ref[b:b + 1, :L] = jnp.where(cols == last, last_ts, row)
        out_ts_ref[b:b + 1, L:L + 1] = row[:, 0:1]
        out_len_ref[b] = last + 1


def kernel(x_payload, timestamps, seq_lens, token):
    seq_lens = seq_lens.astype(jnp.int32)
    token2 = token.reshape(1, D)

    w = jnp.arange(NW, dtype=jnp.int32)
    params = jnp.zeros((NW * 8, 16), jnp.int32)
    params = params.at[w * 8, 0].set(seq_lens[w // 2])

    mesh = plsc.VectorSubcoreMesh(core_axis_name="c", subcore_axis_name="s")
    sc_copy = functools.partial(
        pl.kernel,
        out_type=jax.ShapeDtypeStruct((B, L + 1, D), x_payload.dtype),
        mesh=mesh,
        scratch_types=[
            pltpu.VMEM((2, CHR, D), jnp.float32),
            pltpu.VMEM((1, D), jnp.float32),
            pltpu.VMEM((8, 16), jnp.int32),
            pltpu.SemaphoreType.DMA((2,)),
            pltpu.SemaphoreType.DMA((2,)),
            pltpu.SemaphoreType.DMA,
        ],
    )(_sc_body)
    new_x = sc_copy(x_payload, token2, params)

    new_ts, new_len = pl.pallas_call(
        _ts_body,
        in_specs=[
            pl.BlockSpec(memory_space=pltpu.SMEM),
            pl.BlockSpec(memory_space=pltpu.VMEM),
        ],
        out_specs=[
            pl.BlockSpec(memory_space=pltpu.VMEM),
            pl.BlockSpec(memory_space=pltpu.SMEM),
        ],
        out_shape=[
            jax.ShapeDtypeStruct((B, L + 1), timestamps.dtype),
            jax.ShapeDtypeStruct((B,), jnp.int32),
        ],
    )(seq_lens, timestamps)
    return new_x, new_len, new_ts, new_len


# speedup vs baseline: 1.5064x; 1.0026x over previous
import functools

import jax
import jax.numpy as jnp
from jax import lax
from jax.experimental import pallas as pl
from jax.experimental.pallas import tpu as pltpu
from jax.experimental.pallas import tpu_sc as plsc

B, L, D = 16, 4096, 1024
NC, NS = 2, 16
NW = NC * NS
HALF = L // 2
CHR = 32
NCHUNK = HALF // CHR


def _sc_body(x_hbm, tok_hbm, params_hbm, out_hbm,
             buf, row_v, prow_v, sem_in, sem_out, sem_m):
    wid = lax.axis_index("s") * NC + lax.axis_index("c")
    b = wid // 2
    half = wid % 2
    row0 = half * HALF

    pltpu.sync_copy(
        params_hbm.at[pl.ds(pl.multiple_of(wid * 8, 8), 8)], prow_v)
    last = prow_v[0][0]
    own_tok = (last >= row0) & (last < row0 + HALF)
    offs = last - row0
    tok_chunk = offs // CHR
    tok_rloc = offs % CHR

    @pl.when(half == 1)
    def _wrap():
        pltpu.async_copy(x_hbm.at[b, pl.ds(0, 1)], row_v, sem_m).wait()
        pltpu.async_copy(row_v, out_hbm.at[b, pl.ds(L, 1)], sem_m).wait()

    loads = [None, None]
    stores = [None, None]
    loads[0] = pltpu.async_copy(
        x_hbm.at[b, pl.ds(pl.multiple_of(row0, 8), CHR)], buf.at[0],
        sem_in.at[0])
    for i in range(NCHUNK):
        cur = i & 1
        nxt = cur ^ 1
        if i < NCHUNK - 1:
            if i >= 1:
                stores[nxt].wait()
            loads[nxt] = pltpu.async_copy(
                x_hbm.at[b, pl.ds(pl.multiple_of(row0 + (i + 1) * CHR, 8),
                                  CHR)],
                buf.at[nxt], sem_in.at[nxt])
        loads[cur].wait()

        @pl.when(own_tok & (tok_chunk == i))
        def _patch(cur=cur):
            pltpu.sync_copy(tok_hbm, buf.at[cur, pl.ds(tok_rloc, 1)])

        stores[cur] = pltpu.async_copy(
            buf.at[cur],
            out_hbm.at[b, pl.ds(pl.multiple_of(row0 + i * CHR, 8), CHR)],
            sem_out.at[cur])
    stores[0].wait()
    stores[1].wait()


def _ts_body(lens_ref, ts_ref, out_ts_ref, out_len_ref):
    cols = lax.broadcasted_iota(jnp.int32, (1, L), 1)
    for b in range(B):
        last = lens_ref[b]
        last_m1 = jnp.maximum(last - 1, 0)
        row = ts_ref[b:b + 1, :]
        last_ts = jnp.sum(jnp.where(cols == last_m1, row, 0.0))
        out_ts_ref[b:b + 1, :L] = jnp.where(cols == last, last_ts, row)
        out_ts_ref[b:b + 1, L:L + 1] = row[:, 0:1]
        out_len_ref[b] = last + 1


def kernel(x_payload, timestamps, seq_lens, token):
    seq_lens = seq_lens.astype(jnp.int32)
    token2 = token.reshape(1, D)

    w = jnp.arange(NW, dtype=jnp.int32)
    params = jnp.zeros((NW * 8, 16), jnp.int32)
    params = params.at[w * 8, 0].set(seq_lens[w // 2])

    mesh = plsc.VectorSubcoreMesh(core_axis_name="c", subcore_axis_name="s")
    sc_copy = functools.partial(
        pl.kernel,
        out_type=jax.ShapeDtypeStruct((B, L + 1, D), x_payload.dtype),
        mesh=mesh,
        scratch_types=[
            pltpu.VMEM((2, CHR, D), jnp.float32),
            pltpu.VMEM((1, D), jnp.float32),
            pltpu.VMEM((8, 16), jnp.int32),
            pltpu.SemaphoreType.DMA((2,)),
            pltpu.SemaphoreType.DMA((2,)),
            pltpu.SemaphoreType.DMA,
        ],
        compiler_params=pltpu.CompilerParams(use_tc_tiling_on_sc=True),
    )(_sc_body)
    new_x = sc_copy(x_payload, token2, params)

    new_ts, new_len = pl.pallas_call(
        _ts_body,
        in_specs=[
            pl.BlockSpec(memory_space=pltpu.SMEM),
            pl.BlockSpec(memory_space=pltpu.VMEM),
        ],
        out_specs=[
            pl.BlockSpec(memory_space=pltpu.VMEM),
            pl.BlockSpec(memory_space=pltpu.SMEM),
        ],
        out_shape=[
            jax.ShapeDtypeStruct((B, L + 1), timestamps.dtype),
            jax.ShapeDtypeStruct((B,), jnp.int32),
        ],
    )(seq_lens, timestamps)
    return new_x, new_len, new_ts, new_len


# speedup vs baseline: 2.9893x; 1.9844x over previous
import functools

import jax
import jax.numpy as jnp
from jax import lax
from jax.experimental import pallas as pl
from jax.experimental.pallas import tpu as pltpu
from jax.experimental.pallas import tpu_sc as plsc

B, L, D = 16, 4096, 1024
NC, NS = 2, 16
NW = NC * NS
SLAB = L // NW
CHRJ = 2
CHUNK = CHRJ * B
NCHUNK = SLAB // CHRJ


def _sc_body(x_hbm, tok_hbm, params_hbm, out_hbm,
             vbuf, pbuf, prow_v, idx2_v, sem_in, sem_out, sem_m):
    wid = lax.axis_index("s") * NC + lax.axis_index("c")
    j0 = wid * SLAB

    pltpu.sync_copy(
        params_hbm.at[pl.ds(pl.multiple_of(wid * 8, 8), 8)], prow_v)
    lens = prow_v[0]

    lane = lax.broadcasted_iota(jnp.int32, (NS,), 0)
    in_lane = lane * L

    def set_idx(cur, jrow):
        idx2_v[cur, 0:NS] = in_lane + jrow
        idx2_v[cur, NS:2 * NS] = in_lane + (jrow + 1)

    loads = [None, None]
    stores = [None, None]
    set_idx(0, j0)
    loads[0] = pltpu.async_copy(
        x_hbm.at[idx2_v.at[0]], vbuf.at[pl.ds(0, CHUNK)], sem_in.at[0])
    for i in range(NCHUNK):
        cur = i & 1
        nxt = cur ^ 1
        if i < NCHUNK - 1:
            if i >= 1:
                stores[nxt].wait()
            set_idx(nxt, j0 + (i + 1) * CHRJ)
            loads[nxt] = pltpu.async_copy(
                x_hbm.at[idx2_v.at[nxt]],
                vbuf.at[pl.ds(nxt * CHUNK, CHUNK)], sem_in.at[nxt])
        loads[cur].wait()
        stores[cur] = pltpu.async_copy(
            vbuf.at[pl.ds(cur * CHUNK, CHUNK)],
            out_hbm.at[pl.ds(pl.multiple_of((j0 + i * CHRJ) * B, 8), CHUNK)],
            sem_out.at[cur])
    stores[0].wait()
    stores[1].wait()

    for b in range(B):
        last_b = lens[b]

        @pl.when((last_b >= j0) & (last_b < j0 + SLAB))
        def _patch(last_b=last_b, b=b):
            blk = out_hbm.at[pl.ds(pl.multiple_of(last_b * B, 8), B)]
            pltpu.async_copy(blk, pbuf, sem_m).wait()
            pltpu.sync_copy(tok_hbm, pbuf.at[pl.ds(b, 1)])
            pltpu.async_copy(pbuf, blk, sem_m).wait()

    @pl.when(wid == NW - 1)
    def _wrap():
        for b in range(B):
            pltpu.sync_copy(x_hbm.at[pl.ds(b * L, 1)],
                            pbuf.at[pl.ds(b, 1)])
        pltpu.async_copy(pbuf, out_hbm.at[pl.ds(L * B, B)], sem_m).wait()


def _ts_body(lens_ref, ts_ref, out_ts_ref, out_len_ref):
    cols = lax.broadcasted_iota(jnp.int32, (1, L), 1)
    for b in range(B):
        last = lens_ref[b]
        last_m1 = jnp.maximum(last - 1, 0)
        row = ts_ref[b:b + 1, :]
        last_ts = jnp.sum(jnp.where(cols == last_m1, row, 0.0))
        out_ts_ref[b:b + 1, :L] = jnp.where(cols == last, last_ts, row)
        out_ts_ref[b:b + 1, L:L + 1] = row[:, 0:1]
        out_len_ref[b] = last + 1


def kernel(x_payload, timestamps, seq_lens, token):
    seq_lens = seq_lens.astype(jnp.int32)
    token2 = token.reshape(1, D)

    params = jnp.zeros((NW * 8, 16), jnp.int32)
    params = params.at[jnp.arange(NW) * 8].set(
        jnp.broadcast_to(seq_lens, (NW, B)))

    x_flat = x_payload.reshape(B * L, D)

    mesh = plsc.VectorSubcoreMesh(core_axis_name="c", subcore_axis_name="s")
    sc_copy = functools.partial(
        pl.kernel,
        out_type=jax.ShapeDtypeStruct(((L + 1) * B, D), x_payload.dtype),
        mesh=mesh,
        scratch_types=[
            pltpu.VMEM((2 * CHUNK, D), jnp.float32),
            pltpu.VMEM((B, D), jnp.float32),
            pltpu.VMEM((8, 16), jnp.int32),
            pltpu.VMEM((2, CHUNK), jnp.int32),
            pltpu.SemaphoreType.DMA((2,)),
            pltpu.SemaphoreType.DMA((2,)),
            pltpu.SemaphoreType.DMA,
        ],
    )(_sc_body)
    out_flat = sc_copy(x_flat, token2, params)
    new_x = out_flat.reshape(L + 1, B, D).transpose(1, 0, 2)

    new_ts, new_len = pl.pallas_call(
        _ts_body,
        in_specs=[
            pl.BlockSpec(memory_space=pltpu.SMEM),
            pl.BlockSpec(memory_space=pltpu.VMEM),
        ],
        out_specs=[
            pl.BlockSpec(memory_space=pltpu.VMEM),
            pl.BlockSpec(memory_space=pltpu.SMEM),
        ],
        out_shape=[
            jax.ShapeDtypeStruct((B, L + 1), timestamps.dtype),
            jax.ShapeDtypeStruct((B,), jnp.int32),
        ],
    )(seq_lens, timestamps)
    return new_x, new_len, new_ts, new_len
